# Initial kernel scaffold; baseline (speedup 1.0000x reference)
import jax
import jax.numpy as jnp
from jax.experimental import pallas as pl


def kernel(state, node_features, edge_feature, edge_index, params_input, params_message, params_update, params_output):
    raise NotImplementedError("write your pallas kernel here")


# speedup vs baseline: 1.9612x; 1.9612x over previous
import functools

import jax
import jax.numpy as jnp
from jax.experimental import pallas as pl
from jax.experimental.pallas import tpu as pltpu
from jax.experimental.pallas import tpu_sc as plsc

N = 10000
E = 160000
NP = 10240
EP = 163840
W = 128
TILES = 32
PT = EP // TILES
CH = 128
NCH = PT // CH
NB_N = NP // 1024
NB_E = EP // 1024
NUM_MP = 6


@functools.lru_cache(maxsize=None)
def _sc_mesh():
    return plsc.VectorSubcoreMesh(core_axis_name="c", subcore_axis_name="s")


def _ln_relu(x, g, b):
    m = jnp.mean(x, axis=-1, keepdims=True)
    d = x - m
    v = jnp.mean(d * d, axis=-1, keepdims=True)
    return jnp.maximum(d / jnp.sqrt(v + 1e-5) * g + b, 0.0)


def _dot(a, b):
    return jnp.dot(a.astype(jnp.bfloat16), b.astype(jnp.bfloat16),
                   preferred_element_type=jnp.float32)


def _full(spec_shape):
    return pl.BlockSpec(spec_shape, lambda i: tuple(0 for _ in spec_shape))


def _input_mlp_body(x_ref, w1, b1, g1, e1, w2, b2, g2, e2, w3, b3, o_ref):
    h = _ln_relu(_dot(x_ref[:, 0:32], w1[...]) + b1[...], g1[...], e1[...])
    h = _ln_relu(_dot(h, w2[...]) + b2[...], g2[...], e2[...])
    o_ref[:, 0:64] = _dot(h, w3[...]) + b3[...]
    o_ref[:, 64:W] = jnp.zeros((1024, W - 64), jnp.float32)


def _input_mlp(x, w1, b1, g1, e1, w2, b2, g2, e2, w3, b3):
    return pl.pallas_call(
        _input_mlp_body,
        grid=(NB_N,),
        in_specs=[
            pl.BlockSpec((1024, W), lambda i: (i, 0)),
            _full((32, 256)), _full((1, 256)), _full((1, 256)), _full((1, 256)),
            _full((256, 256)), _full((1, 256)), _full((1, 256)), _full((1, 256)),
            _full((256, 64)), _full((1, 64)),
        ],
        out_specs=pl.BlockSpec((1024, W), lambda i: (i, 0)),
        out_shape=jax.ShapeDtypeStruct((NP, W), jnp.float32),
    )(x, w1, b1, g1, e1, w2, b2, g2, e2, w3, b3)


def _edge_mlp_body(s_ref, c_ref, w1s, w1c, b1, g1, e1, w2, b2, g2, e2, w3,
                   b3, o_ref):
    z = (_dot(s_ref[:, 0:64], w1s[...]) + _dot(c_ref[:, 0:32], w1c[...])
         + b1[...])
    h = _ln_relu(z, g1[...], e1[...])
    h = _ln_relu(_dot(h, w2[...]) + b2[...], g2[...], e2[...])
    o_ref[:, 0:64] = jnp.tanh(_dot(h, w3[...]) + b3[...])
    o_ref[:, 64:65] = jnp.ones((1024, 1), jnp.float32)
    o_ref[:, 65:W] = jnp.zeros((1024, W - 65), jnp.float32)


def _edge_mlp(s, c, w1s, w1c, b1, g1, e1, w2, b2, g2, e2, w3, b3):
    return pl.pallas_call(
        _edge_mlp_body,
        grid=(NB_E,),
        in_specs=[
            pl.BlockSpec((1024, W), lambda i: (i, 0)),
            pl.BlockSpec((1024, W), lambda i: (i, 0)),
            _full((64, 256)), _full((32, 256)), _full((1, 256)),
            _full((1, 256)), _full((1, 256)),
            _full((256, 256)), _full((1, 256)), _full((1, 256)), _full((1, 256)),
            _full((256, 64)), _full((1, 64)),
        ],
        out_specs=pl.BlockSpec((1024, W), lambda i: (i, 0)),
        out_shape=jax.ShapeDtypeStruct((EP, W), jnp.float32),
    )(s, c, w1s, w1c, b1, g1, e1, w2, b2, g2, e2, w3, b3)


def _update_mlp_body(p_ref, s_ref, w1m, w1s, b1, g1, e1, w2, b2, g2, e2, w3,
                     b3, o_ref):
    deg = p_ref[0, :, 64:65] + p_ref[1, :, 64:65]
    inv = 1.0 / jnp.maximum(deg, 1.0)
    mh = (p_ref[0, :, 0:64] + p_ref[1, :, 0:64]) * inv
    z = _dot(mh, w1m[...]) + _dot(s_ref[:, 0:64], w1s[...]) + b1[...]
    h = _ln_relu(z, g1[...], e1[...])
    h = _ln_relu(_dot(h, w2[...]) + b2[...], g2[...], e2[...])
    o_ref[:, 0:64] = _dot(h, w3[...]) + b3[...]
    o_ref[:, 64:W] = jnp.zeros((1024, W - 64), jnp.float32)


def _update_mlp(p, s, w1m, w1s, b1, g1, e1, w2, b2, g2, e2, w3, b3):
    return pl.pallas_call(
        _update_mlp_body,
        grid=(NB_N,),
        in_specs=[
            pl.BlockSpec((2, 1024, W), lambda i: (0, i, 0)),
            pl.BlockSpec((1024, W), lambda i: (i, 0)),
            _full((64, 256)), _full((64, 256)), _full((1, 256)),
            _full((1, 256)), _full((1, 256)),
            _full((256, 256)), _full((1, 256)), _full((1, 256)), _full((1, 256)),
            _full((256, 64)), _full((1, 64)),
        ],
        out_specs=pl.BlockSpec((1024, W), lambda i: (i, 0)),
        out_shape=jax.ShapeDtypeStruct((NP, W), jnp.float32),
    )(p, s, w1m, w1s, b1, g1, e1, w2, b2, g2, e2, w3, b3)


def _output_mlp_body(s_ref, w1, b1, g1, e1, w2, b2, g2, e2, wo, bo, ws, bs,
                     o_ref, sum_ref):
    i = pl.program_id(0)
    h = _ln_relu(_dot(s_ref[:, 0:64], w1[...]) + b1[...], g1[...], e1[...])
    h = _ln_relu(_dot(h, w2[...]) + b2[...], g2[...], e2[...])
    act = jnp.tanh(_dot(h, wo[...]) + bo[...])
    sg = _dot(h, ws[...]) + bs[...]
    sg = 1.0 / (1.0 + jnp.exp(-sg))
    o_ref[:, 0:1] = act
    o_ref[:, 1:2] = sg
    rid = jax.lax.broadcasted_iota(jnp.int32, (1024, 1), 0) + i * 1024
    part = jnp.sum(jnp.where(rid < N, sg, 0.0), axis=(0, 1), keepdims=True)

    @pl.when(i == 0)
    def _():
        sum_ref[...] = jnp.zeros((1, 1), jnp.float32)

    sum_ref[...] += part

    @pl.when(i == NB_N - 1)
    def _():
        sum_ref[...] = sum_ref[...] * (1.0 / N)


def _output_mlp(s, w1, b1, g1, e1, w2, b2, g2, e2, wo, bo, ws, bs):
    return pl.pallas_call(
        _output_mlp_body,
        grid=(NB_N,),
        in_specs=[
            pl.BlockSpec((1024, W), lambda i: (i, 0)),
            _full((64, 256)), _full((1, 256)), _full((1, 256)), _full((1, 256)),
            _full((256, 256)), _full((1, 256)), _full((1, 256)), _full((1, 256)),
            _full((256, 1)), _full((1, 1)), _full((256, 1)), _full((1, 1)),
        ],
        out_specs=[
            pl.BlockSpec((1024, 2), lambda i: (i, 0)),
            pl.BlockSpec((1, 1), lambda i: (0, 0)),
        ],
        out_shape=[
            jax.ShapeDtypeStruct((NP, 2), jnp.float32),
            jax.ShapeDtypeStruct((1, 1), jnp.float32),
        ],
    )(s, w1, b1, g1, e1, w2, b2, g2, e2, wo, bo, ws, bs)


@functools.lru_cache(maxsize=None)
def _make_gather():
    def body(tab_hbm, idx_hbm, out_hbm, idx_v, rows_v, sem0, sem1):
        c = jax.lax.axis_index("c")
        s = jax.lax.axis_index("s")
        wid = s * 2 + c
        base = wid * PT
        pltpu.sync_copy(idx_hbm.at[wid], idx_v)
        sems = (sem0, sem1)

        def start(j, buf):
            pltpu.make_async_copy(
                tab_hbm.at[idx_v.at[j]], rows_v.at[buf], sems[buf]).start()

        def wait(j, buf):
            pltpu.make_async_copy(
                tab_hbm.at[idx_v.at[j]], rows_v.at[buf], sems[buf]).wait()

        def put(j, buf):
            pltpu.sync_copy(rows_v.at[buf],
                            out_hbm.at[pl.ds(base + j * CH, CH)])

        start(0, 0)

        def loop(k, carry):
            j = k * 2
            start(j + 1, 1)
            wait(j, 0)
            put(j, 0)

            @pl.when(k < NCH // 2 - 1)
            def _():
                start(j + 2, 0)

            wait(j + 1, 1)
            put(j + 1, 1)
            return carry

        jax.lax.fori_loop(0, NCH // 2, loop, 0)

    return pl.kernel(
        body,
        out_type=jax.ShapeDtypeStruct((EP, W), jnp.float32),
        mesh=_sc_mesh(),
        scratch_types=[
            pltpu.VMEM((NCH, CH), jnp.int32),
            pltpu.VMEM((2, CH, W), jnp.float32),
            pltpu.SemaphoreType.DMA,
            pltpu.SemaphoreType.DMA,
        ],
    )


@functools.lru_cache(maxsize=None)
def _make_scatter():
    rpt = NP // 16

    def body(msg_hbm, idx_hbm, zero_hbm, out_hbm, idx_v, rows_v, acc_sh,
             sem0, sem1):
        c = jax.lax.axis_index("c")
        s = jax.lax.axis_index("s")
        wid = s * 2 + c
        base = wid * PT
        pltpu.sync_copy(zero_hbm.at[pl.ds(s * rpt, rpt)],
                        acc_sh.at[pl.ds(s * rpt, rpt)])
        pltpu.sync_copy(idx_hbm.at[wid], idx_v)
        plsc.subcore_barrier()
        sems = (sem0, sem1)

        def start(j, buf):
            pltpu.make_async_copy(
                msg_hbm.at[pl.ds(base + j * CH, CH)], rows_v.at[buf],
                sems[buf]).start()

        def wait(j, buf):
            pltpu.make_async_copy(
                msg_hbm.at[pl.ds(base + j * CH, CH)], rows_v.at[buf],
                sems[buf]).wait()

        def add(j, buf):
            pltpu.sync_copy(rows_v.at[buf], acc_sh.at[idx_v.at[j]], add=True)

        start(0, 0)

        def loop(k, carry):
            j = k * 2
            start(j + 1, 1)
            wait(j, 0)
            add(j, 0)

            @pl.when(k < NCH // 2 - 1)
            def _():
                start(j + 2, 0)

            wait(j + 1, 1)
            add(j + 1, 1)
            return carry

        jax.lax.fori_loop(0, NCH // 2, loop, 0)
        plsc.subcore_barrier()
        pltpu.sync_copy(acc_sh.at[pl.ds(s * rpt, rpt)],
                        out_hbm.at[c, pl.ds(s * rpt, rpt)])

    return pl.kernel(
        body,
        out_type=jax.ShapeDtypeStruct((2, NP, W), jnp.float32),
        mesh=_sc_mesh(),
        scratch_types=[
            pltpu.VMEM((NCH, CH), jnp.int32),
            pltpu.VMEM((2, CH, W), jnp.float32),
            pltpu.VMEM_SHARED((NP, W), jnp.float32),
            pltpu.SemaphoreType.DMA,
            pltpu.SemaphoreType.DMA,
        ],
    )


def _vecs(p, i):
    w, b, g, e = p["hidden"][i]
    return w, b.reshape(1, -1), g.reshape(1, -1), e.reshape(1, -1)


@jax.jit
def _run(state, node_features, edge_feature, edge_index, pi, pm, pu, po):
    nSV = state.shape[1] // 2

    glob = jnp.concatenate([state[:, 0:5], state[:, nSV:nSV + 5]], axis=-1)
    cols = [
        node_features,
        jnp.broadcast_to(glob, (N, 10)),
        state[0, 5:5 + N][:, None],
        state[0, 5 + N:5 + 2 * N][:, None],
        state[0, nSV + 5:nSV + 5 + N][:, None],
        state[0, nSV + 5 + N:nSV + 5 + 2 * N][:, None],
    ]
    ni = jnp.concatenate(cols, axis=1)
    ni_pad = jnp.pad(ni, ((0, NP - N), (0, W - 20)))

    src = jnp.pad(edge_index[0], (0, EP - E))
    dst = jnp.pad(edge_index[1], (0, EP - E), constant_values=N)
    src3 = src.reshape(TILES, NCH, CH)
    dst3 = dst.reshape(TILES, NCH, CH)
    efeat = jnp.pad(edge_feature, (0, EP - E))

    w1i, b1i, g1i, e1i = _vecs(pi, 0)
    w1i = jnp.pad(w1i, ((0, 12), (0, 0)))
    w2i, b2i, g2i, e2i = _vecs(pi, 1)
    w1m, b1m, g1m, e1m = _vecs(pm, 0)
    w1m_s = w1m[0:64]
    w1m_c = jnp.pad(jnp.concatenate([w1m[65:85], w1m[64:65]], axis=0),
                    ((0, 11), (0, 0)))
    w2m, b2m, g2m, e2m = _vecs(pm, 1)
    w1u, b1u, g1u, e1u = _vecs(pu, 0)
    w1u_m, w1u_s = w1u[0:64], w1u[64:128]
    w2u, b2u, g2u, e2u = _vecs(pu, 1)
    w1o, b1o, g1o, e1o = _vecs(po, 0)
    w2o, b2o, g2o, e2o = _vecs(po, 1)

    ns = _input_mlp(ni_pad, w1i, b1i, g1i, e1i, w2i, b2i, g2i, e2i,
                    pi["Wout"], pi["bout"].reshape(1, -1))

    cmat = _make_gather()(ni_pad, src3)
    cmat = cmat.at[:, 20].set(efeat)

    zero = jnp.zeros((NP, W), jnp.float32)

    for _ in range(NUM_MP):
        se = _make_gather()(ns, src3)
        msg = _edge_mlp(se, cmat, w1m_s, w1m_c, b1m, g1m, e1m, w2m, b2m,
                        g2m, e2m, pm["Wout"], pm["bout"].reshape(1, -1))
        part = _make_scatter()(msg, dst3, zero)
        ns = _update_mlp(part, ns, w1u_m, w1u_s, b1u, g1u, e1u, w2u,
                         b2u, g2u, e2u, pu["Wout"], pu["bout"].reshape(1, -1))

    out2, ssum = _output_mlp(ns, w1o, b1o, g1o, e1o, w2o, b2o, g2o, e2o,
                             po["Wout"], po["bout"].reshape(1, -1),
                             po["Wsig"], po["bsig"].reshape(1, -1))
    actions = out2[:N, 0][None, :]
    sigmoids = ssum.reshape(1)
    return actions, sigmoids


def kernel(state, node_features, edge_feature, edge_index, params_input,
           params_message, params_update, params_output):
    return _run(state, node_features, edge_feature, edge_index, params_input,
                params_message, params_update, params_output)
